# baseline (speedup 1.0000x reference)
import functools
import jax
import jax.numpy as jnp
from jax import lax
from jax.experimental import pallas as pl
from jax.experimental.pallas import tpu as pltpu
from jax.experimental.pallas import tpu_sc as plsc

EMBED = 2048
NLEV = 16
L = 16
NW = 32
N = 1024 * 20
RPW = N // NW
GPW = RPW // L


def _encode_body(w_ref, t_ref, u_ref):
    w = w_ref[...]
    t = t_ref[...]
    ws = w[:-1, :]
    we = w[1:, :]
    u_ref[...] = ws * jnp.where(ws == we, -1.5, t)


def _encode(weight, threshold):
    return pl.pallas_call(
        _encode_body,
        out_shape=jax.ShapeDtypeStruct((NLEV - 1, EMBED), jnp.float32),
    )(weight, threshold)


def _sc_body(x_hbm, u_hbm, out_hbm, x_v, u_v, buf_v):
    cid = lax.axis_index("c")
    sid = lax.axis_index("s")
    wid = sid * 2 + cid
    base = wid * RPW
    pltpu.sync_copy(u_hbm, u_v)
    pltpu.sync_copy(x_hbm.at[pl.ds(base * 1, RPW)], x_v)
    lanes = lax.iota(jnp.int32, L)
    st0 = lanes * EMBED

    def group(g, carry):
        xv = x_v[pl.ds(g * L, L)]
        span = jnp.clip(xv * float(NLEV - 1), 0.0, float(NLEV - 1))
        idxi = jnp.minimum(span.astype(jnp.int32), NLEV - 2)
        tau = span - idxi.astype(jnp.float32)
        ga0 = idxi * EMBED
        dvec = st0 - ga0

        def col(c, ga):
            u16 = plsc.load_gather(u_v, [ga])
            ub = plsc.bitcast(u16, jnp.int32)
            neg = ub < 0
            hit = tau > jnp.abs(u16)
            val = jnp.where(neg != hit, 1.0, -1.0).astype(jnp.float32)
            plsc.store_scatter(buf_v, [ga + dvec], val)
            return ga + 1

        lax.fori_loop(0, EMBED, col, ga0)
        pltpu.sync_copy(buf_v, out_hbm.at[pl.ds((base + g * L) * EMBED, L * EMBED)])
        return carry

    lax.fori_loop(0, GPW, group, 0)


@jax.jit
def _run(x_flat, u_flat):
    mesh = plsc.VectorSubcoreMesh(core_axis_name="c", subcore_axis_name="s")
    sc = pl.kernel(
        _sc_body,
        out_type=jax.ShapeDtypeStruct((N * EMBED,), jnp.float32),
        mesh=mesh,
        compiler_params=pltpu.CompilerParams(needs_layout_passes=False),
        scratch_types=[
            pltpu.VMEM((RPW,), jnp.float32),
            pltpu.VMEM(((NLEV - 1) * EMBED,), jnp.float32),
            pltpu.VMEM((L * EMBED,), jnp.float32),
        ],
    )
    return sc(x_flat, u_flat)


def kernel(input, weight, threshold):
    u = _encode(weight, threshold)
    out = _run(input.reshape(N), u.reshape((NLEV - 1) * EMBED))
    return out.reshape(*input.shape, EMBED)


# speedup vs baseline: 6.7841x; 6.7841x over previous
import functools
import jax
import jax.numpy as jnp
from jax import lax
from jax.experimental import pallas as pl
from jax.experimental.pallas import tpu as pltpu
from jax.experimental.pallas import tpu_sc as plsc

EMBED = 2048
NLEV = 16
L = 16
NW = 32
B0 = 1024
B1 = 20
N = B0 * B1
BPW = B0 // NW
CHUNK = EMBED // L


def _encode_body(w_ref, t_ref, u_ref):
    w = w_ref[...]
    t = t_ref[...]
    ws = w[:-1, :]
    we = w[1:, :]
    u_ref[...] = ws * jnp.where(ws == we, -1.5, t)


def _encode(weight, threshold):
    return pl.pallas_call(
        _encode_body,
        out_shape=jax.ShapeDtypeStruct((NLEV - 1, EMBED), jnp.float32),
    )(weight, threshold)


def _sc_body(x_hbm, u_hbm, out_hbm, x_v, u_v, tau_v, gb_v, buf_v, sem0, sem1):
    cid = lax.axis_index("c")
    sid = lax.axis_index("s")
    wid = sid * 2 + cid
    base_b = wid * BPW
    pltpu.sync_copy(u_hbm, u_v)
    pltpu.sync_copy(x_hbm.at[pl.ds(base_b * B1, BPW * B1)], x_v)
    lanes = lax.iota(jnp.int32, L)

    def pre(j, carry):
        xv = x_v[pl.ds(j * L, L)]
        span = jnp.clip(xv * float(NLEV - 1), 0.0, float(NLEV - 1))
        idxi = jnp.minimum(span.astype(jnp.int32), NLEV - 2)
        tau_v[pl.ds(j * L, L)] = span - idxi.astype(jnp.float32)
        gb_v[pl.ds(j * L, L)] = idxi * EMBED
        return carry

    lax.fori_loop(0, (BPW * B1) // L, pre, 0)

    def batch(g, carry):
        slot = lax.rem(g, 2)
        @pl.when(g >= 2)
        def _wait():
            @pl.when(slot == 0)
            def _w0():
                pltpu.make_async_copy(buf_v.at[0], out_hbm.at[base_b], sem0).wait()

            @pl.when(slot == 1)
            def _w1():
                pltpu.make_async_copy(buf_v.at[1], out_hbm.at[base_b], sem1).wait()

        def row(r, carry2):
            rsplat = jnp.full((L,), g * B1 + r, jnp.int32)
            tau = plsc.load_gather(tau_v, [rsplat])
            ga0 = plsc.load_gather(gb_v, [rsplat]) + lanes

            UNR = 8

            def col(j, ga):
                us = [plsc.load_gather(u_v, [ga + (k * L)]) for k in range(UNR)]
                for k in range(UNR):
                    u16 = us[k]
                    ub = plsc.bitcast(u16, jnp.int32)
                    neg = ub < 0
                    hit = tau > jnp.abs(u16)
                    val = jnp.where(neg != hit, 1.0, -1.0).astype(jnp.float32)
                    buf_v[slot, r, pl.ds(j * (UNR * L) + k * L, L)] = val
                return ga + UNR * L

            lax.fori_loop(0, CHUNK // UNR, col, ga0)
            return carry2

        lax.fori_loop(0, B1, row, 0)

        @pl.when(slot == 0)
        def _s0():
            pltpu.async_copy(buf_v.at[0], out_hbm.at[base_b + g], sem0)

        @pl.when(slot == 1)
        def _s1():
            pltpu.async_copy(buf_v.at[1], out_hbm.at[base_b + g], sem1)

        return carry

    lax.fori_loop(0, BPW, batch, 0)
    pltpu.make_async_copy(buf_v.at[0], out_hbm.at[base_b], sem0).wait()
    pltpu.make_async_copy(buf_v.at[1], out_hbm.at[base_b], sem1).wait()


@jax.jit
def _run(x_flat, u_flat):
    mesh = plsc.VectorSubcoreMesh(core_axis_name="c", subcore_axis_name="s")
    sc = pl.kernel(
        _sc_body,
        out_type=jax.ShapeDtypeStruct((B0, B1, EMBED), jnp.float32),
        mesh=mesh,
        compiler_params=pltpu.CompilerParams(
            needs_layout_passes=False,
            use_tc_tiling_on_sc=True,
        ),
        scratch_types=[
            pltpu.VMEM((BPW * B1,), jnp.float32),
            pltpu.VMEM(((NLEV - 1) * EMBED,), jnp.float32),
            pltpu.VMEM((BPW * B1,), jnp.float32),
            pltpu.VMEM((BPW * B1,), jnp.int32),
            pltpu.VMEM((2, B1, EMBED), jnp.float32),
            pltpu.SemaphoreType.DMA,
            pltpu.SemaphoreType.DMA,
        ],
    )
    return sc(x_flat, u_flat)


def kernel(input, weight, threshold):
    u = _encode(weight, threshold)
    out = _run(input.reshape(N), u.reshape((NLEV - 1) * EMBED))
    return out.reshape(*input.shape, EMBED)


# speedup vs baseline: 12.0306x; 1.7734x over previous
import functools
import jax
import jax.numpy as jnp
from jax import lax
from jax.experimental import pallas as pl
from jax.experimental.pallas import tpu as pltpu
from jax.experimental.pallas import tpu_sc as plsc

EMBED = 2048
NLEV = 16
L = 16
NW = 32
B0 = 1024
B1 = 20
N = B0 * B1
BPW = B0 // NW
CHUNK = EMBED // L


def _encode_body(w_ref, t_ref, u_ref):
    w = w_ref[...]
    t = t_ref[...]
    ws = w[:-1, :]
    we = w[1:, :]
    u_ref[...] = ws * jnp.where(ws == we, -1.5, t)


def _encode(weight, threshold):
    return pl.pallas_call(
        _encode_body,
        out_shape=jax.ShapeDtypeStruct((NLEV - 1, EMBED), jnp.float32),
    )(weight, threshold)


def _sc_body(x_hbm, u_hbm, out_hbm, x_v, u_v, tau_v, gb_v, buf_v, sem0, sem1):
    cid = lax.axis_index("c")
    sid = lax.axis_index("s")
    wid = sid * 2 + cid
    base_b = wid * BPW
    pltpu.sync_copy(u_hbm, u_v.at[pl.ds(0, (NLEV - 1) * EMBED)])
    pltpu.sync_copy(x_hbm.at[pl.ds(base_b * B1, BPW * B1)], x_v)
    lanes = lax.iota(jnp.int32, L)

    def pre(j, carry):
        xv = x_v[pl.ds(j * L, L)]
        span = jnp.clip(xv * float(NLEV - 1), 0.0, float(NLEV - 1))
        idxi = jnp.minimum(span.astype(jnp.int32), NLEV - 2)
        tau_v[pl.ds(j * L, L)] = span - idxi.astype(jnp.float32)
        gb_v[pl.ds(j * L, L)] = idxi * EMBED
        return carry

    lax.fori_loop(0, (BPW * B1) // L, pre, 0)

    u_wins = [u_v.at[pl.ds(k * L, (NLEV - 1) * EMBED)] for k in range(8)]

    def batch(g, carry):
        j = lax.div(g, 2)
        h = lax.rem(g, 2)
        slot = lax.rem(g, 2)
        @pl.when(g >= 2)
        def _wait():
            @pl.when(slot == 0)
            def _w0():
                pltpu.make_async_copy(buf_v.at[0], out_hbm.at[0, pl.ds(base_b, L)], sem0).wait()

            @pl.when(slot == 1)
            def _w1():
                pltpu.make_async_copy(buf_v.at[1], out_hbm.at[0, pl.ds(base_b, L)], sem1).wait()

        def row(r, carry2):
            rsplat = jnp.full((L,), (h * L + r) * B1 + j, jnp.int32)
            tau = plsc.load_gather(tau_v, [rsplat])
            ga0 = plsc.load_gather(gb_v, [rsplat]) + lanes

            UNR = 8

            def col(j, ga):
                us = [plsc.load_gather(u_wins[k], [ga]) for k in range(UNR)]
                for k in range(UNR):
                    u16 = us[k]
                    ub = plsc.bitcast(u16, jnp.int32)
                    neg = ub < 0
                    hit = tau > jnp.abs(u16)
                    val = jnp.where(neg != hit, 1.0, -1.0).astype(jnp.float32)
                    buf_v[slot, r, pl.ds(j * (UNR * L) + k * L, L)] = val
                return ga + UNR * L

            lax.fori_loop(0, CHUNK // UNR, col, ga0)
            return carry2

        lax.fori_loop(0, L, row, 0)
        dst_row = base_b + h * L

        @pl.when(slot == 0)
        def _s0():
            pltpu.async_copy(buf_v.at[0], out_hbm.at[j, pl.ds(dst_row, L)], sem0)

        @pl.when(slot == 1)
        def _s1():
            pltpu.async_copy(buf_v.at[1], out_hbm.at[j, pl.ds(dst_row, L)], sem1)

        return carry

    lax.fori_loop(0, 2 * B1, batch, 0)
    pltpu.make_async_copy(buf_v.at[0], out_hbm.at[0, pl.ds(base_b, L)], sem0).wait()
    pltpu.make_async_copy(buf_v.at[1], out_hbm.at[0, pl.ds(base_b, L)], sem1).wait()


@jax.jit
def _run(x_flat, u_flat):
    mesh = plsc.VectorSubcoreMesh(core_axis_name="c", subcore_axis_name="s")
    sc = pl.kernel(
        _sc_body,
        out_type=jax.ShapeDtypeStruct((B1, B0, EMBED), jnp.float32),
        mesh=mesh,
        compiler_params=pltpu.CompilerParams(
            needs_layout_passes=False,
            use_tc_tiling_on_sc=True,
        ),
        scratch_types=[
            pltpu.VMEM((BPW * B1,), jnp.float32),
            pltpu.VMEM(((NLEV - 1) * EMBED + 8 * L,), jnp.float32),
            pltpu.VMEM((BPW * B1,), jnp.float32),
            pltpu.VMEM((BPW * B1,), jnp.int32),
            pltpu.VMEM((2, L, EMBED), jnp.float32),
            pltpu.SemaphoreType.DMA,
            pltpu.SemaphoreType.DMA,
        ],
    )
    return sc(x_flat, u_flat)


def kernel(input, weight, threshold):
    u = _encode(weight, threshold)
    out_t = _run(input.reshape(N), u.reshape((NLEV - 1) * EMBED))
    return jnp.transpose(out_t, (1, 0, 2)).reshape(*input.shape, EMBED)
